# Initial kernel scaffold; baseline (speedup 1.0000x reference)
import jax
import jax.numpy as jnp
from jax.experimental import pallas as pl


def kernel(points, normals):
    raise NotImplementedError("write your pallas kernel here")


# speedup vs baseline: 37.1265x; 37.1265x over previous
import functools

import jax
import jax.numpy as jnp
from jax import lax
from jax.experimental import pallas as pl
from jax.experimental.pallas import tpu as pltpu
from jax.experimental.pallas import tpu_sc as plsc

_NN_K = 5
_FILTER_SCALE = 2.0
_SIGMA = 0.75
_EPS = 1e-10
_B, _P, _D = 4, 4096, 3
_BR = 512
_NB = _P // _BR

_NTILES = 32
_PPT = _B * _P // _NTILES
_TPB = _P // _PPT
_LANES = 16


def _knn_tc_body(rows_ref, cols_ref, nrows_ref, d_ref, i_ref, nh_ref):
    rows = rows_ref[0]
    xr, yr, zr = rows[:, 0:1], rows[:, 1:2], rows[:, 2:3]
    cols = cols_ref[0]
    xc, yc, zc = cols[0:1, :], cols[1:2, :], cols[2:3, :]
    sq_r = xr * xr + yr * yr + zr * zr
    sq_c = xc * xc + yc * yc + zc * zc
    pq = jax.lax.dot_general(
        rows.astype(jnp.bfloat16), cols.astype(jnp.bfloat16),
        (((1,), (0,)), ((), ())),
        preferred_element_type=jnp.float32)
    d2 = jnp.maximum(sq_r + sq_c - 2.0 * pq, 0.0)

    iota = lax.broadcasted_iota(jnp.int32, (_BR, _P), 1)
    work = d2
    for k in range(_NN_K + 1):
        m = jnp.min(work, axis=1, keepdims=True)
        idx = jnp.min(jnp.where(work <= m, iota, _P), axis=1,
                      keepdims=True)
        if k >= 1:
            d_ref[0, :, k - 1:k] = m
            i_ref[0, :, k - 1:k] = idx
        work = jnp.where(iota == idx, jnp.float32(jnp.inf), work)

    nr = nrows_ref[0]
    nrm = jnp.sqrt(jnp.sum(nr * nr, axis=1, keepdims=True))
    nh_ref[0] = nr * (1.0 / jnp.where(nrm < _EPS, _EPS, nrm))


def _knn_tc(points, points_t, normals):
    return pl.pallas_call(
        _knn_tc_body,
        grid=(_B, _NB),
        in_specs=[
            pl.BlockSpec((1, _BR, _D), lambda b, rb: (b, rb, 0)),
            pl.BlockSpec((1, _D, _P), lambda b, rb: (b, 0, 0)),
            pl.BlockSpec((1, _BR, _D), lambda b, rb: (b, rb, 0)),
        ],
        out_specs=[
            pl.BlockSpec((1, _BR, _NN_K), lambda b, rb: (b, rb, 0)),
            pl.BlockSpec((1, _BR, _NN_K), lambda b, rb: (b, rb, 0)),
            pl.BlockSpec((1, _BR, _D), lambda b, rb: (b, rb, 0)),
        ],
        out_shape=[
            jax.ShapeDtypeStruct((_B, _P, _NN_K), jnp.float32),
            jax.ShapeDtypeStruct((_B, _P, _NN_K), jnp.int32),
            jax.ShapeDtypeStruct((_B, _P, _D), jnp.float32),
        ],
    )(points, points_t, normals)


_N = _B * _P


def _weights_sc_body(d_hbm, i_hbm, nh_hbm, p_hbm, out_hbm, *scratch):
    ntabs = scratch[0:3]
    ptabs = scratch[3:6]
    owns = scratch[6:9]
    ownp = scratch[9:12]
    dks = scratch[12:17]
    iks = scratch[17:22]
    out_v = scratch[22]

    wid = lax.axis_index("s") * 2 + lax.axis_index("c")
    base = wid * _PPT
    b_off = (wid // _TPB) * _P

    for r in range(_D):
        pltpu.sync_copy(nh_hbm.at[pl.ds(r * _N + b_off, _P)], ntabs[r])
        pltpu.sync_copy(nh_hbm.at[pl.ds(r * _N + base, _PPT)], owns[r])
        pltpu.sync_copy(p_hbm.at[pl.ds(r * _N + b_off, _P)], ptabs[r])
        pltpu.sync_copy(p_hbm.at[pl.ds(r * _N + base, _PPT)], ownp[r])
    for k in range(_NN_K):
        pltpu.sync_copy(d_hbm.at[pl.ds(k * _N + base, _PPT)], dks[k])
        pltpu.sync_copy(i_hbm.at[pl.ds(k * _N + base, _PPT)], iks[k])

    inv_sig_n = 1.0 / (_SIGMA * _SIGMA)

    def chunk(i, carry):
        sl = pl.ds(i * _LANES, _LANES)
        ox, oy, oz = owns[0][sl], owns[1][sl], owns[2][sl]
        px, py, pz = ownp[0][sl], ownp[1][sl], ownp[2][sl]
        d1 = dks[0][sl]
        s = d1 * (2.0 * _FILTER_SCALE * _FILTER_SCALE)
        s = jnp.where(s < _EPS, jnp.float32(_EPS), s)
        inv_sp = 1.0 / jnp.where(d1 < _EPS, jnp.float32(_EPS), d1)
        acc = jnp.zeros((_LANES,), jnp.float32)
        for k in range(_NN_K):
            dk = dks[k][sl]
            idx = iks[k][sl]
            gx = plsc.load_gather(ntabs[0], [idx])
            gy = plsc.load_gather(ntabs[1], [idx])
            gz = plsc.load_gather(ntabs[2], [idx])
            qx = plsc.load_gather(ptabs[0], [idx])
            qy = plsc.load_gather(ptabs[1], [idx])
            qz = plsc.load_gather(ptabs[2], [idx])
            w = jnp.maximum(1.0 - dk / s, 0.0)
            w = w * w
            w = w * w
            dx, dy, dz = gx - ox, gy - oy, gz - oz
            dn2 = dx * dx + dy * dy + dz * dz
            wn = jnp.exp(-dn2 * inv_sig_n)
            ux, uy, uz = qx - px, qy - py, qz - pz
            dp2 = ux * ux + uy * uy + uz * uz
            ws = jnp.exp(-dp2 * inv_sp)
            acc = acc + w * wn * ws * dk
        out_v[sl] = acc
        return carry

    lax.fori_loop(0, _PPT // _LANES, chunk, 0)
    pltpu.sync_copy(out_v, out_hbm.at[pl.ds(base, _PPT)])


def _weights_sc(d5, i5, nh, pf):
    mesh = plsc.VectorSubcoreMesh(core_axis_name="c", subcore_axis_name="s")
    f = pl.kernel(
        _weights_sc_body,
        out_type=jax.ShapeDtypeStruct((_N,), jnp.float32),
        mesh=mesh,
        compiler_params=pltpu.CompilerParams(needs_layout_passes=False),
        scratch_types=(
            [pltpu.VMEM((_P,), jnp.float32)] * 6
            + [pltpu.VMEM((_PPT,), jnp.float32)] * 6
            + [pltpu.VMEM((_PPT,), jnp.float32)] * _NN_K
            + [pltpu.VMEM((_PPT,), jnp.int32)] * _NN_K
            + [pltpu.VMEM((_PPT,), jnp.float32)]
        ),
    )
    return f(d5, i5, nh, pf)


def kernel(points, normals):
    points_t = jnp.transpose(points, (0, 2, 1))
    d5, i5, nh = _knn_tc(points, points_t, normals)
    d5f = jnp.reshape(jnp.transpose(d5, (2, 0, 1)), (_NN_K * _N,))
    i5f = jnp.reshape(jnp.transpose(i5, (2, 0, 1)), (_NN_K * _N,))
    nhf = jnp.reshape(jnp.transpose(nh, (2, 0, 1)), (_D * _N,))
    pf = jnp.reshape(jnp.transpose(points, (2, 0, 1)), (_D * _N,))
    per_point = _weights_sc(d5f, i5f, nhf, pf)
    return jnp.mean(per_point)


# speedup vs baseline: 44.6048x; 1.2014x over previous
import functools

import jax
import jax.numpy as jnp
from jax import lax
from jax.experimental import pallas as pl
from jax.experimental.pallas import tpu as pltpu
from jax.experimental.pallas import tpu_sc as plsc

_NN_K = 5
_FILTER_SCALE = 2.0
_SIGMA = 0.75
_EPS = 1e-10
_B, _P, _D = 4, 4096, 3
_BR = 512
_NB = _P // _BR

_NTILES = 32
_PPT = _B * _P // _NTILES
_TPB = _P // _PPT
_LANES = 16


def _knn_tc_body(rows_ref, cols_ref, nrows_ref, d_ref, i_ref, nh_ref):
    rows = rows_ref[0]
    xr, yr, zr = rows[:, 0:1], rows[:, 1:2], rows[:, 2:3]
    cols = cols_ref[0]
    xc, yc, zc = cols[0:1, :], cols[1:2, :], cols[2:3, :]
    sq_r = xr * xr + yr * yr + zr * zr
    sq_c = xc * xc + yc * yc + zc * zc
    pq = jax.lax.dot_general(
        rows.astype(jnp.bfloat16), cols.astype(jnp.bfloat16),
        (((1,), (0,)), ((), ())),
        preferred_element_type=jnp.float32)
    d2 = jnp.maximum(sq_r + sq_c - 2.0 * pq, 0.0)

    half = _P // 2
    a = d2[:, :half]
    b = d2[:, half:]
    ia = lax.broadcasted_iota(jnp.int32, (_BR, half), 1).astype(jnp.float32)
    ib = ia + jnp.float32(half)
    amask = a <= b
    work = jnp.minimum(a, b)
    cur = jnp.where(amask, ia, ib)
    lval = jnp.maximum(a, b)
    lidx = jnp.where(amask, ib, ia)
    big = jnp.float32(2.0 * _P)
    inf = jnp.float32(jnp.inf)
    for k in range(_NN_K + 1):
        m = jnp.min(work, axis=1, keepdims=True)
        idxf = jnp.min(jnp.where(work <= m, cur, big), axis=1,
                       keepdims=True)
        if k >= 1:
            d_ref[0, :, k - 1:k] = m
            i_ref[0, :, k - 1:k] = idxf.astype(jnp.int32)
        e = cur == idxf
        work = jnp.where(e, lval, work)
        cur = jnp.where(e, lidx, cur)
        lval = jnp.where(e, inf, lval)

    nr = nrows_ref[0]
    nrm = jnp.sqrt(jnp.sum(nr * nr, axis=1, keepdims=True))
    nh_ref[0] = nr * (1.0 / jnp.where(nrm < _EPS, _EPS, nrm))


def _knn_tc(points, points_t, normals):
    return pl.pallas_call(
        _knn_tc_body,
        grid=(_B, _NB),
        in_specs=[
            pl.BlockSpec((1, _BR, _D), lambda b, rb: (b, rb, 0)),
            pl.BlockSpec((1, _D, _P), lambda b, rb: (b, 0, 0)),
            pl.BlockSpec((1, _BR, _D), lambda b, rb: (b, rb, 0)),
        ],
        out_specs=[
            pl.BlockSpec((1, _BR, _NN_K), lambda b, rb: (b, rb, 0)),
            pl.BlockSpec((1, _BR, _NN_K), lambda b, rb: (b, rb, 0)),
            pl.BlockSpec((1, _BR, _D), lambda b, rb: (b, rb, 0)),
        ],
        out_shape=[
            jax.ShapeDtypeStruct((_B, _P, _NN_K), jnp.float32),
            jax.ShapeDtypeStruct((_B, _P, _NN_K), jnp.int32),
            jax.ShapeDtypeStruct((_B, _P, _D), jnp.float32),
        ],
    )(points, points_t, normals)


_N = _B * _P


def _weights_sc_body(d_hbm, i_hbm, nh_hbm, p_hbm, out_hbm, *scratch):
    ntabs = scratch[0:3]
    ptabs = scratch[3:6]
    owns = scratch[6:9]
    ownp = scratch[9:12]
    dks = scratch[12:17]
    iks = scratch[17:22]
    out_v = scratch[22]

    wid = lax.axis_index("s") * 2 + lax.axis_index("c")
    base = wid * _PPT
    b_off = (wid // _TPB) * _P

    for r in range(_D):
        pltpu.sync_copy(nh_hbm.at[pl.ds(r * _N + b_off, _P)], ntabs[r])
        pltpu.sync_copy(nh_hbm.at[pl.ds(r * _N + base, _PPT)], owns[r])
        pltpu.sync_copy(p_hbm.at[pl.ds(r * _N + b_off, _P)], ptabs[r])
        pltpu.sync_copy(p_hbm.at[pl.ds(r * _N + base, _PPT)], ownp[r])
    for k in range(_NN_K):
        pltpu.sync_copy(d_hbm.at[pl.ds(k * _N + base, _PPT)], dks[k])
        pltpu.sync_copy(i_hbm.at[pl.ds(k * _N + base, _PPT)], iks[k])

    inv_sig_n = 1.0 / (_SIGMA * _SIGMA)

    def chunk(i, carry):
        sl = pl.ds(i * _LANES, _LANES)
        ox, oy, oz = owns[0][sl], owns[1][sl], owns[2][sl]
        px, py, pz = ownp[0][sl], ownp[1][sl], ownp[2][sl]
        d1 = dks[0][sl]
        s = d1 * (2.0 * _FILTER_SCALE * _FILTER_SCALE)
        s = jnp.where(s < _EPS, jnp.float32(_EPS), s)
        inv_sp = 1.0 / jnp.where(d1 < _EPS, jnp.float32(_EPS), d1)
        acc = jnp.zeros((_LANES,), jnp.float32)
        for k in range(_NN_K):
            dk = dks[k][sl]
            idx = iks[k][sl]
            gx = plsc.load_gather(ntabs[0], [idx])
            gy = plsc.load_gather(ntabs[1], [idx])
            gz = plsc.load_gather(ntabs[2], [idx])
            qx = plsc.load_gather(ptabs[0], [idx])
            qy = plsc.load_gather(ptabs[1], [idx])
            qz = plsc.load_gather(ptabs[2], [idx])
            w = jnp.maximum(1.0 - dk / s, 0.0)
            w = w * w
            w = w * w
            dx, dy, dz = gx - ox, gy - oy, gz - oz
            dn2 = dx * dx + dy * dy + dz * dz
            wn = jnp.exp(-dn2 * inv_sig_n)
            ux, uy, uz = qx - px, qy - py, qz - pz
            dp2 = ux * ux + uy * uy + uz * uz
            ws = jnp.exp(-dp2 * inv_sp)
            acc = acc + w * wn * ws * dk
        out_v[sl] = acc
        return carry

    lax.fori_loop(0, _PPT // _LANES, chunk, 0)
    pltpu.sync_copy(out_v, out_hbm.at[pl.ds(base, _PPT)])


def _weights_sc(d5, i5, nh, pf):
    mesh = plsc.VectorSubcoreMesh(core_axis_name="c", subcore_axis_name="s")
    f = pl.kernel(
        _weights_sc_body,
        out_type=jax.ShapeDtypeStruct((_N,), jnp.float32),
        mesh=mesh,
        compiler_params=pltpu.CompilerParams(needs_layout_passes=False),
        scratch_types=(
            [pltpu.VMEM((_P,), jnp.float32)] * 6
            + [pltpu.VMEM((_PPT,), jnp.float32)] * 6
            + [pltpu.VMEM((_PPT,), jnp.float32)] * _NN_K
            + [pltpu.VMEM((_PPT,), jnp.int32)] * _NN_K
            + [pltpu.VMEM((_PPT,), jnp.float32)]
        ),
    )
    return f(d5, i5, nh, pf)


def kernel(points, normals):
    points_t = jnp.transpose(points, (0, 2, 1))
    d5, i5, nh = _knn_tc(points, points_t, normals)
    d5f = jnp.reshape(jnp.transpose(d5, (2, 0, 1)), (_NN_K * _N,))
    i5f = jnp.reshape(jnp.transpose(i5, (2, 0, 1)), (_NN_K * _N,))
    nhf = jnp.reshape(jnp.transpose(nh, (2, 0, 1)), (_D * _N,))
    pf = jnp.reshape(jnp.transpose(points, (2, 0, 1)), (_D * _N,))
    per_point = _weights_sc(d5f, i5f, nhf, pf)
    return jnp.mean(per_point)
